# baseline (speedup 1.0000x reference)
import functools

import jax
import jax.numpy as jnp
from jax.experimental import pallas as pl

BF = jnp.bfloat16
F32 = jnp.float32
EPS = 1e-5


def _xw_body(x_ref, w_ref, o_ref):
    o_ref[...] = jnp.dot(x_ref[...], w_ref[...],
                         preferred_element_type=F32).astype(BF)


def _hw_body(h_ref, wcat_ref, o_ref):
    o_ref[...] = jnp.dot(h_ref[...].astype(BF), wcat_ref[...],
                         preferred_element_type=F32).astype(BF)


def _z_body(adj_ref, hw_ref, bcat_ref, zm_ref, zd_ref, zp_ref):
    d = zm_ref.shape[1]
    a = adj_ref[...].astype(BF)
    z = jax.lax.dot_general(
        a, hw_ref[...], (((1,), (0,)), ((), ())), preferred_element_type=F32)
    z = z + bcat_ref[...]
    zm_ref[...] = jnp.exp(z[:, :d])
    zd_ref[...] = jax.nn.sigmoid(z[:, d:2 * d])
    zp_ref[...] = jnp.exp(z[:, 2 * d:3 * d])


def _md_body(z_ref, w_ref, b_ref, o_ref):
    i = pl.program_id(0)

    @pl.when(i == 0)
    def _():
        o_ref[...] = jnp.zeros_like(o_ref)

    d = jnp.dot(z_ref[...], w_ref[...], preferred_element_type=F32)
    o_ref[...] += jnp.sum(d + b_ref[...], axis=0, keepdims=True)


def _var_body(z_ref, w_ref, b_ref, md_ref, o_ref, *, n_rows):
    i = pl.program_id(0)

    @pl.when(i == 0)
    def _():
        o_ref[...] = jnp.zeros_like(o_ref)

    d = jnp.dot(z_ref[...], w_ref[...], preferred_element_type=F32) + b_ref[...]
    dc = d - md_ref[...] / n_rows
    o_ref[...] += jnp.sum(dc * dc, axis=0, keepdims=True)


def _dec_body(z_ref, md_ref, vs_ref, w1_ref, b1d_ref, g_ref, b_ref, w2_ref,
              b2_ref, sf_ref, o_ref, *, n_rows):
    d = jnp.dot(z_ref[...], w1_ref[...], preferred_element_type=F32) + b1d_ref[...]
    dc = d - md_ref[...] / n_rows
    dn = jnp.maximum(
        g_ref[...] * dc / jnp.sqrt(vs_ref[...] / n_rows + EPS) + b_ref[...],
        0.0)
    dec = jnp.maximum(
        jnp.dot(dn, w2_ref[...], preferred_element_type=F32) + b2_ref[...],
        0.0)
    o_ref[...] = jnp.maximum(sf_ref[...] * dec, 0.0)


def kernel(x, adj, x_t, size_factors, W1, b1, gn_w, gn_b, gn_ms, Wm, bm, Wd,
           bd, Wp, bp, fc1_w, fc1_b, bn2_g, bn2_b, fc2_w, fc2_b):
    n, d = x.shape
    p1 = W1.shape[1]
    p2 = fc1_w.shape[1]

    bi = 200
    bv = 1000

    wcat = jnp.concatenate([Wm, Wd, Wp], axis=1).astype(BF)
    bcat = jnp.concatenate([bm, bd, bp]).reshape(1, 3 * d)
    bn2gr = bn2_g.reshape(1, p2)
    bn2br = bn2_b.reshape(1, p2)
    fc1br = fc1_b.reshape(1, p2)
    fc2br = fc2_b.reshape(1, d)

    xw = (x @ W1).astype(BF)
    h_pre = adj.T @ xw.astype(F32) + b1
    mu = h_pre.mean(axis=0)
    hc = h_pre - gn_ms * mu
    var = (hc * hc).mean(axis=0)
    h = jax.nn.relu(gn_w * hc / jnp.sqrt(var + EPS) + gn_b)

    hw = pl.pallas_call(
        _hw_body,
        out_shape=jax.ShapeDtypeStruct((n, 3 * d), BF),
    )(h, wcat)

    nblk = n // bi
    zm, zd, zp = pl.pallas_call(
        _z_body,
        grid=(nblk,),
        in_specs=[
            pl.BlockSpec((bi, n), lambda i: (i, 0)),
            pl.BlockSpec((n, 3 * d), lambda i: (0, 0)),
            pl.BlockSpec((1, 3 * d), lambda i: (0, 0)),
        ],
        out_specs=[
            pl.BlockSpec((bi, d), lambda i: (i, 0)),
            pl.BlockSpec((bi, d), lambda i: (i, 0)),
            pl.BlockSpec((bi, d), lambda i: (i, 0)),
        ],
        out_shape=[
            jax.ShapeDtypeStruct((n, d), F32),
            jax.ShapeDtypeStruct((n, d), F32),
            jax.ShapeDtypeStruct((n, d), F32),
        ],
    )(adj, hw, bcat)

    nbv = n // bv
    md = pl.pallas_call(
        _md_body,
        grid=(nbv,),
        in_specs=[
            pl.BlockSpec((bv, d), lambda i: (i, 0)),
            pl.BlockSpec((d, p2), lambda i: (0, 0)),
            pl.BlockSpec((1, p2), lambda i: (0, 0)),
        ],
        out_specs=pl.BlockSpec((1, p2), lambda i: (0, 0)),
        out_shape=jax.ShapeDtypeStruct((1, p2), F32),
    )(zm, fc1_w, fc1br)

    vs = pl.pallas_call(
        functools.partial(_var_body, n_rows=float(n)),
        grid=(nbv,),
        in_specs=[
            pl.BlockSpec((bv, d), lambda i: (i, 0)),
            pl.BlockSpec((d, p2), lambda i: (0, 0)),
            pl.BlockSpec((1, p2), lambda i: (0, 0)),
            pl.BlockSpec((1, p2), lambda i: (0, 0)),
        ],
        out_specs=pl.BlockSpec((1, p2), lambda i: (0, 0)),
        out_shape=jax.ShapeDtypeStruct((1, p2), F32),
    )(zm, fc1_w, fc1br, md)

    xr = pl.pallas_call(
        functools.partial(_dec_body, n_rows=float(n)),
        grid=(nbv,),
        in_specs=[
            pl.BlockSpec((bv, d), lambda i: (i, 0)),
            pl.BlockSpec((1, p2), lambda i: (0, 0)),
            pl.BlockSpec((1, p2), lambda i: (0, 0)),
            pl.BlockSpec((d, p2), lambda i: (0, 0)),
            pl.BlockSpec((1, p2), lambda i: (0, 0)),
            pl.BlockSpec((1, p2), lambda i: (0, 0)),
            pl.BlockSpec((1, p2), lambda i: (0, 0)),
            pl.BlockSpec((p2, d), lambda i: (0, 0)),
            pl.BlockSpec((1, d), lambda i: (0, 0)),
            pl.BlockSpec((bv, 1), lambda i: (i, 0)),
        ],
        out_specs=pl.BlockSpec((bv, d), lambda i: (i, 0)),
        out_shape=jax.ShapeDtypeStruct((n, d), F32),
    )(zm, md, vs, fc1_w, fc1br, bn2gr, bn2br, fc2_w, fc2br, size_factors)

    return (xr, zm, zd, zp)


# speedup vs baseline: 1.0524x; 1.0524x over previous
import functools

import jax
import jax.numpy as jnp
from jax.experimental import pallas as pl

BF = jnp.bfloat16
F32 = jnp.float32
EPS = 1e-5


def _hw_body(h_ref, wcat_ref, o_ref):
    o_ref[...] = jnp.dot(h_ref[...].astype(BF), wcat_ref[...],
                         preferred_element_type=F32).astype(BF)


def _z_body(adj_ref, hw_ref, bcat_ref, zm_ref, zd_ref, zp_ref):
    d = zm_ref.shape[1]
    a = adj_ref[...].astype(BF)
    z = jax.lax.dot_general(
        a, hw_ref[...], (((1,), (0,)), ((), ())), preferred_element_type=F32)
    z = z + bcat_ref[...]
    zm_ref[...] = jnp.exp(z[:, :d])
    zd_ref[...] = jax.nn.sigmoid(z[:, d:2 * d])
    zp_ref[...] = jnp.exp(z[:, 2 * d:3 * d])


def _md_body(z_ref, w_ref, b_ref, o_ref):
    i = pl.program_id(0)

    @pl.when(i == 0)
    def _():
        o_ref[...] = jnp.zeros_like(o_ref)

    d = jnp.dot(z_ref[...], w_ref[...], preferred_element_type=F32)
    o_ref[...] += jnp.sum(d + b_ref[...], axis=0, keepdims=True)


def _var_body(z_ref, w_ref, b_ref, md_ref, o_ref, *, n_rows):
    i = pl.program_id(0)

    @pl.when(i == 0)
    def _():
        o_ref[...] = jnp.zeros_like(o_ref)

    d = jnp.dot(z_ref[...], w_ref[...], preferred_element_type=F32) + b_ref[...]
    dc = d - md_ref[...] / n_rows
    o_ref[...] += jnp.sum(dc * dc, axis=0, keepdims=True)


def _dec_body(z_ref, md_ref, vs_ref, w1_ref, b1d_ref, g_ref, b_ref, w2_ref,
              b2_ref, sf_ref, o_ref, *, n_rows):
    d = jnp.dot(z_ref[...], w1_ref[...], preferred_element_type=F32) + b1d_ref[...]
    dc = d - md_ref[...] / n_rows
    dn = jnp.maximum(
        g_ref[...] * dc / jnp.sqrt(vs_ref[...] / n_rows + EPS) + b_ref[...],
        0.0)
    dec = jnp.maximum(
        jnp.dot(dn, w2_ref[...], preferred_element_type=F32) + b2_ref[...],
        0.0)
    o_ref[...] = jnp.maximum(sf_ref[...] * dec, 0.0)


def kernel(x, adj, x_t, size_factors, W1, b1, gn_w, gn_b, gn_ms, Wm, bm, Wd,
           bd, Wp, bp, fc1_w, fc1_b, bn2_g, bn2_b, fc2_w, fc2_b):
    n, d = x.shape
    p1 = W1.shape[1]
    p2 = fc1_w.shape[1]

    bi = 400
    bv = 1000

    wcat = jnp.concatenate([Wm, Wd, Wp], axis=1).astype(BF)
    bcat = jnp.concatenate([bm, bd, bp]).reshape(1, 3 * d)
    bn2gr = bn2_g.reshape(1, p2)
    bn2br = bn2_b.reshape(1, p2)
    fc1br = fc1_b.reshape(1, p2)
    fc2br = fc2_b.reshape(1, d)

    xw = (x @ W1).astype(BF)
    h_pre = adj.T @ xw.astype(F32) + b1
    mu = h_pre.mean(axis=0)
    hc = h_pre - gn_ms * mu
    var = (hc * hc).mean(axis=0)
    h = jax.nn.relu(gn_w * hc / jnp.sqrt(var + EPS) + gn_b)

    hw = pl.pallas_call(
        _hw_body,
        out_shape=jax.ShapeDtypeStruct((n, 3 * d), BF),
    )(h, wcat)

    nblk = n // bi
    zm, zd, zp = pl.pallas_call(
        _z_body,
        grid=(nblk,),
        in_specs=[
            pl.BlockSpec((bi, n), lambda i: (i, 0)),
            pl.BlockSpec((n, 3 * d), lambda i: (0, 0)),
            pl.BlockSpec((1, 3 * d), lambda i: (0, 0)),
        ],
        out_specs=[
            pl.BlockSpec((bi, d), lambda i: (i, 0)),
            pl.BlockSpec((bi, d), lambda i: (i, 0)),
            pl.BlockSpec((bi, d), lambda i: (i, 0)),
        ],
        out_shape=[
            jax.ShapeDtypeStruct((n, d), F32),
            jax.ShapeDtypeStruct((n, d), F32),
            jax.ShapeDtypeStruct((n, d), F32),
        ],
    )(adj, hw, bcat)

    nbv = n // bv
    md = pl.pallas_call(
        _md_body,
        grid=(nbv,),
        in_specs=[
            pl.BlockSpec((bv, d), lambda i: (i, 0)),
            pl.BlockSpec((d, p2), lambda i: (0, 0)),
            pl.BlockSpec((1, p2), lambda i: (0, 0)),
        ],
        out_specs=pl.BlockSpec((1, p2), lambda i: (0, 0)),
        out_shape=jax.ShapeDtypeStruct((1, p2), F32),
    )(zm, fc1_w, fc1br)

    vs = pl.pallas_call(
        functools.partial(_var_body, n_rows=float(n)),
        grid=(nbv,),
        in_specs=[
            pl.BlockSpec((bv, d), lambda i: (i, 0)),
            pl.BlockSpec((d, p2), lambda i: (0, 0)),
            pl.BlockSpec((1, p2), lambda i: (0, 0)),
            pl.BlockSpec((1, p2), lambda i: (0, 0)),
        ],
        out_specs=pl.BlockSpec((1, p2), lambda i: (0, 0)),
        out_shape=jax.ShapeDtypeStruct((1, p2), F32),
    )(zm, fc1_w, fc1br, md)

    xr = pl.pallas_call(
        functools.partial(_dec_body, n_rows=float(n)),
        grid=(nbv,),
        in_specs=[
            pl.BlockSpec((bv, d), lambda i: (i, 0)),
            pl.BlockSpec((1, p2), lambda i: (0, 0)),
            pl.BlockSpec((1, p2), lambda i: (0, 0)),
            pl.BlockSpec((d, p2), lambda i: (0, 0)),
            pl.BlockSpec((1, p2), lambda i: (0, 0)),
            pl.BlockSpec((1, p2), lambda i: (0, 0)),
            pl.BlockSpec((1, p2), lambda i: (0, 0)),
            pl.BlockSpec((p2, d), lambda i: (0, 0)),
            pl.BlockSpec((1, d), lambda i: (0, 0)),
            pl.BlockSpec((bv, 1), lambda i: (i, 0)),
        ],
        out_specs=pl.BlockSpec((bv, d), lambda i: (i, 0)),
        out_shape=jax.ShapeDtypeStruct((n, d), F32),
    )(zm, md, vs, fc1_w, fc1br, bn2gr, bn2br, fc2_w, fc2br, size_factors)

    return (xr, zm, zd, zp)
